# Initial kernel scaffold; baseline (speedup 1.0000x reference)
import jax
import jax.numpy as jnp
from jax.experimental import pallas as pl


def kernel(x, edge_index, edge_weight, W1, b1, W2, b2):
    raise NotImplementedError("write your pallas kernel here")


# speedup vs baseline: 4.1682x; 4.1682x over previous
import functools

import jax
import jax.numpy as jnp
from jax import lax
from jax.experimental import pallas as pl
from jax.experimental.pallas import tpu as pltpu
from jax.experimental.pallas import tpu_sc as plsc

N = 10000
E = 320000
D = 128

NC = 2
NS = 16
NW = NC * NS
EP = E // NW
C = 80
NCH = EP // C
RPT = 624
ZR = 104
TAIL = N - NS * RPT


def _mm_body(x_ref, w_ref, o_ref):
    o_ref[...] = jnp.dot(x_ref[...], w_ref[...],
                         preferred_element_type=jnp.float32)


def _matmul(x, W, bm=2000):
    n, k = x.shape
    m = W.shape[1]
    return pl.pallas_call(
        _mm_body,
        grid=(n // bm,),
        in_specs=[pl.BlockSpec((bm, k), lambda i: (i, 0)),
                  pl.BlockSpec((k, m), lambda i: (0, 0))],
        out_specs=pl.BlockSpec((bm, m), lambda i: (i, 0)),
        out_shape=jax.ShapeDtypeStruct((n, m), jnp.float32),
    )(x, W)


def _fused_body(p0_ref, p1_ref, b_ref, w_ref, o_ref):
    h = jnp.maximum(p0_ref[...] + p1_ref[...] + b_ref[...], 0.0)
    o_ref[...] = jnp.dot(h, w_ref[...], preferred_element_type=jnp.float32)


def _fused_relu_mm(p0, p1, b, W, bm=2000):
    n, k = p0.shape
    m = W.shape[1]
    return pl.pallas_call(
        _fused_body,
        grid=(n // bm,),
        in_specs=[pl.BlockSpec((bm, k), lambda i: (i, 0)),
                  pl.BlockSpec((bm, k), lambda i: (i, 0)),
                  pl.BlockSpec((1, k), lambda i: (0, 0)),
                  pl.BlockSpec((k, m), lambda i: (0, 0))],
        out_specs=pl.BlockSpec((bm, m), lambda i: (i, 0)),
        out_shape=jax.ShapeDtypeStruct((n, m), jnp.float32),
    )(p0, p1, b.reshape(1, k), W)


def _bias_body(p0_ref, p1_ref, b_ref, o_ref):
    o_ref[...] = p0_ref[...] + p1_ref[...] + b_ref[...]


def _add_partials_bias(p0, p1, b, bm=2000):
    n, k = p0.shape
    return pl.pallas_call(
        _bias_body,
        grid=(n // bm,),
        in_specs=[pl.BlockSpec((bm, k), lambda i: (i, 0)),
                  pl.BlockSpec((bm, k), lambda i: (i, 0)),
                  pl.BlockSpec((1, k), lambda i: (0, 0))],
        out_specs=pl.BlockSpec((bm, k), lambda i: (i, 0)),
        out_shape=jax.ShapeDtypeStruct((n, k), jnp.float32),
    )(p0, p1, b.reshape(1, k))


def _agg_body(h_hbm, src_hbm, dst_hbm, w_hbm, out_hbm,
              srcbuf, dstbuf, wbuf, rowbuf, zbuf, agg, sem):
    c = lax.axis_index("c")
    s = lax.axis_index("s")
    wid = c * NS + s

    def _zrow(r, _):
        zero = jnp.zeros((16,), jnp.float32)
        for j in range(D // 16):
            zbuf[r, pl.ds(j * 16, 16)] = zero
        return _
    lax.fori_loop(0, ZR, _zrow, None)
    row0 = s * RPT
    for i in range(RPT // ZR):
        pltpu.sync_copy(zbuf, agg.at[pl.ds(row0 + i * ZR, ZR)])

    @pl.when(s == NS - 1)
    def _zero_tail():
        pltpu.sync_copy(zbuf.at[pl.ds(0, TAIL)], agg.at[pl.ds(NS * RPT, TAIL)])
    plsc.subcore_barrier()

    def _chunk(g, _):
        base = wid * EP + g * C
        pltpu.sync_copy(src_hbm.at[pl.ds(base, C)], srcbuf)
        pltpu.sync_copy(dst_hbm.at[pl.ds(base, C)], dstbuf)
        pltpu.sync_copy(w_hbm.at[pl.ds(base, C)], wbuf)
        pltpu.async_copy(h_hbm.at[srcbuf], rowbuf, sem).wait()

        def _q(q, _):
            w16 = wbuf[pl.ds(q * 16, 16)]
            for i in range(16):
                wv = jnp.full((16,), w16[i], jnp.float32)
                r = q * 16 + i
                for j in range(D // 16):
                    rowbuf[r, pl.ds(j * 16, 16)] = (
                        rowbuf[r, pl.ds(j * 16, 16)] * wv)
            return _
        lax.fori_loop(0, C // 16, _q, None)

        pltpu.sync_copy(rowbuf, agg.at[dstbuf], add=True)
        return _
    lax.fori_loop(0, NCH, _chunk, None)
    plsc.subcore_barrier()

    for i in range(RPT // ZR):
        r0 = row0 + i * ZR
        pltpu.sync_copy(agg.at[pl.ds(r0, ZR)], out_hbm.at[c, pl.ds(r0, ZR)])

    @pl.when(s == NS - 1)
    def _copy_tail():
        pltpu.sync_copy(agg.at[pl.ds(NS * RPT, TAIL)],
                        out_hbm.at[c, pl.ds(NS * RPT, TAIL)])


def _sc_aggregate(h, src, dst, edge_weight):
    mesh = plsc.VectorSubcoreMesh(core_axis_name="c", subcore_axis_name="s")
    run = pl.kernel(
        _agg_body,
        out_type=jax.ShapeDtypeStruct((NC, N, D), jnp.float32),
        mesh=mesh,
        scratch_types=[
            pltpu.VMEM((C,), jnp.int32),
            pltpu.VMEM((C,), jnp.int32),
            pltpu.VMEM((C,), jnp.float32),
            pltpu.VMEM((C, D), jnp.float32),
            pltpu.VMEM((ZR, D), jnp.float32),
            pltpu.VMEM_SHARED((N, D), jnp.float32),
            pltpu.SemaphoreType.DMA,
        ],
    )
    return run(h, src, dst, edge_weight)


@jax.jit
def kernel(x, edge_index, edge_weight, W1, b1, W2, b2):
    src = edge_index[0]
    dst = edge_index[1]
    h1 = _matmul(x, W1)
    p1 = _sc_aggregate(h1, src, dst, edge_weight)
    h2 = _fused_relu_mm(p1[0], p1[1], b1, W2)
    p2 = _sc_aggregate(h2, src, dst, edge_weight)
    return _add_partials_bias(p2[0], p2[1], b2)


# speedup vs baseline: 4.4172x; 1.0598x over previous
import functools

import jax
import jax.numpy as jnp
from jax import lax
from jax.experimental import pallas as pl
from jax.experimental.pallas import tpu as pltpu
from jax.experimental.pallas import tpu_sc as plsc

N = 10000
E = 320000
D = 128

NC = 2
NS = 16
NW = NC * NS
EP = E // NW
C = 80
NCH = EP // C
RPT = 624
ZR = 104
TAIL = N - NS * RPT


def _mm_body(x_ref, w_ref, o_ref):
    o_ref[...] = jnp.dot(x_ref[...], w_ref[...],
                         preferred_element_type=jnp.float32)


def _matmul(x, W, bm=2000):
    n, k = x.shape
    m = W.shape[1]
    return pl.pallas_call(
        _mm_body,
        grid=(n // bm,),
        in_specs=[pl.BlockSpec((bm, k), lambda i: (i, 0)),
                  pl.BlockSpec((k, m), lambda i: (0, 0))],
        out_specs=pl.BlockSpec((bm, m), lambda i: (i, 0)),
        out_shape=jax.ShapeDtypeStruct((n, m), jnp.float32),
    )(x, W)


def _fused_body(p0_ref, p1_ref, b_ref, w_ref, o_ref):
    h = jnp.maximum(p0_ref[...] + p1_ref[...] + b_ref[...], 0.0)
    o_ref[...] = jnp.dot(h, w_ref[...], preferred_element_type=jnp.float32)


def _fused_relu_mm(p0, p1, b, W, bm=2000):
    n, k = p0.shape
    m = W.shape[1]
    return pl.pallas_call(
        _fused_body,
        grid=(n // bm,),
        in_specs=[pl.BlockSpec((bm, k), lambda i: (i, 0)),
                  pl.BlockSpec((bm, k), lambda i: (i, 0)),
                  pl.BlockSpec((1, k), lambda i: (0, 0)),
                  pl.BlockSpec((k, m), lambda i: (0, 0))],
        out_specs=pl.BlockSpec((bm, m), lambda i: (i, 0)),
        out_shape=jax.ShapeDtypeStruct((n, m), jnp.float32),
    )(p0, p1, b.reshape(1, k), W)


def _bias_body(p0_ref, p1_ref, b_ref, o_ref):
    o_ref[...] = p0_ref[...] + p1_ref[...] + b_ref[...]


def _add_partials_bias(p0, p1, b, bm=2000):
    n, k = p0.shape
    return pl.pallas_call(
        _bias_body,
        grid=(n // bm,),
        in_specs=[pl.BlockSpec((bm, k), lambda i: (i, 0)),
                  pl.BlockSpec((bm, k), lambda i: (i, 0)),
                  pl.BlockSpec((1, k), lambda i: (0, 0))],
        out_specs=pl.BlockSpec((bm, k), lambda i: (i, 0)),
        out_shape=jax.ShapeDtypeStruct((n, k), jnp.float32),
    )(p0, p1, b.reshape(1, k))


RB = 3
IB = 4


def _agg_body(h_hbm, src_hbm, dst_hbm, w_hbm, out_hbm,
              srcbuf, dstbuf, wbuf, rowbuf, zbuf, agg, isem, gsem, ssem):
    c = lax.axis_index("c")
    s = lax.axis_index("s")
    wid = c * NS + s

    def _zrow(r, _):
        zero = jnp.zeros((16,), jnp.float32)
        for j in range(D // 16):
            zbuf[r, pl.ds(j * 16, 16)] = zero
        return _
    lax.fori_loop(0, ZR, _zrow, None)
    row0 = s * RPT
    for i in range(RPT // ZR):
        pltpu.sync_copy(zbuf, agg.at[pl.ds(row0 + i * ZR, ZR)])

    @pl.when(s == NS - 1)
    def _zero_tail():
        pltpu.sync_copy(zbuf.at[pl.ds(0, TAIL)], agg.at[pl.ds(NS * RPT, TAIL)])
    plsc.subcore_barrier()

    def fire_idx(g):
        sl = lax.rem(g, IB)
        base = wid * EP + g * C
        pltpu.async_copy(src_hbm.at[pl.ds(base, C)], srcbuf.at[sl], isem.at[sl])
        pltpu.async_copy(dst_hbm.at[pl.ds(base, C)], dstbuf.at[sl], isem.at[sl])
        pltpu.async_copy(w_hbm.at[pl.ds(base, C)], wbuf.at[sl], isem.at[sl])

    def wait_idx(g):
        sl = lax.rem(g, IB)
        pltpu.make_async_copy(src_hbm.at[pl.ds(0, C)], srcbuf.at[sl],
                              isem.at[sl]).wait()
        pltpu.make_async_copy(dst_hbm.at[pl.ds(0, C)], dstbuf.at[sl],
                              isem.at[sl]).wait()
        pltpu.make_async_copy(w_hbm.at[pl.ds(0, C)], wbuf.at[sl],
                              isem.at[sl]).wait()

    def fire_gather(g):
        ri = lax.rem(g, RB)
        ii = lax.rem(g, IB)
        pltpu.async_copy(h_hbm.at[srcbuf.at[ii]], rowbuf.at[ri], gsem.at[ri])

    def wait_gather(g):
        ri = lax.rem(g, RB)
        ii = lax.rem(g, IB)
        pltpu.make_async_copy(h_hbm.at[srcbuf.at[ii]], rowbuf.at[ri],
                              gsem.at[ri]).wait()

    def fire_scatter(g):
        ri = lax.rem(g, RB)
        ii = lax.rem(g, IB)
        pltpu.async_copy(rowbuf.at[ri], agg.at[dstbuf.at[ii]], ssem.at[ri],
                         add=True)

    def wait_scatter(g):
        ri = lax.rem(g, RB)
        ii = lax.rem(g, IB)
        pltpu.make_async_copy(rowbuf.at[ri], agg.at[dstbuf.at[ii]],
                              ssem.at[ri]).wait()

    fire_idx(0)
    fire_idx(1)
    wait_idx(0)
    fire_gather(0)

    def _chunk(g, _):
        @pl.when(g >= 2)
        def _():
            wait_scatter(g - 2)

        @pl.when(g + 1 < NCH)
        def _():
            wait_idx(g + 1)
            fire_gather(g + 1)

        @pl.when(g + 2 < NCH)
        def _():
            fire_idx(g + 2)

        wait_gather(g)
        ri = lax.rem(g, RB)
        ii = lax.rem(g, IB)

        def _q(q, _):
            w16 = wbuf[ii, pl.ds(q * 16, 16)]
            for i in range(16):
                wv = jnp.full((16,), w16[i], jnp.float32)
                r = q * 16 + i
                for j in range(D // 16):
                    rowbuf[ri, r, pl.ds(j * 16, 16)] = (
                        rowbuf[ri, r, pl.ds(j * 16, 16)] * wv)
            return _
        lax.fori_loop(0, C // 16, _q, None)

        fire_scatter(g)
        return _
    lax.fori_loop(0, NCH, _chunk, None)
    wait_scatter(NCH - 2)
    wait_scatter(NCH - 1)
    plsc.subcore_barrier()

    for i in range(RPT // ZR):
        r0 = row0 + i * ZR
        pltpu.sync_copy(agg.at[pl.ds(r0, ZR)], out_hbm.at[c, pl.ds(r0, ZR)])

    @pl.when(s == NS - 1)
    def _copy_tail():
        pltpu.sync_copy(agg.at[pl.ds(NS * RPT, TAIL)],
                        out_hbm.at[c, pl.ds(NS * RPT, TAIL)])


def _sc_aggregate(h, src, dst, edge_weight):
    mesh = plsc.VectorSubcoreMesh(core_axis_name="c", subcore_axis_name="s")
    run = pl.kernel(
        _agg_body,
        out_type=jax.ShapeDtypeStruct((NC, N, D), jnp.float32),
        mesh=mesh,
        scratch_types=[
            pltpu.VMEM((IB, C), jnp.int32),
            pltpu.VMEM((IB, C), jnp.int32),
            pltpu.VMEM((IB, C), jnp.float32),
            pltpu.VMEM((RB, C, D), jnp.float32),
            pltpu.VMEM((ZR, D), jnp.float32),
            pltpu.VMEM_SHARED((N, D), jnp.float32),
            pltpu.SemaphoreType.DMA((IB,)),
            pltpu.SemaphoreType.DMA((RB,)),
            pltpu.SemaphoreType.DMA((RB,)),
        ],
    )
    return run(h, src, dst, edge_weight)


@jax.jit
def kernel(x, edge_index, edge_weight, W1, b1, W2, b2):
    src = edge_index[0]
    dst = edge_index[1]
    h1 = _matmul(x, W1)
    p1 = _sc_aggregate(h1, src, dst, edge_weight)
    h2 = _fused_relu_mm(p1[0], p1[1], b1, W2)
    p2 = _sc_aggregate(h2, src, dst, edge_weight)
    return _add_partials_bias(p2[0], p2[1], b2)
